# baseline (speedup 1.0000x reference)
import jax
import jax.numpy as jnp
from jax.experimental import pallas as pl
from jax.experimental.pallas import tpu as pltpu

_D = 512
_H = 256
_BLOCK = 2000


def _bilinear_kernel(a1, a2, p1, p2, n1, n2, w_ref, b_ref, out_ref):
    t = jax.lax.dot_general(a1[:], w_ref[:, :_H],
                            (((1,), (1,)), ((), ())),
                            preferred_element_type=jnp.float32)
    t = t + jax.lax.dot_general(a2[:], w_ref[:, _H:],
                                (((1,), (1,)), ((), ())),
                                preferred_element_type=jnp.float32)
    bias = b_ref[0]
    logit_p = (jnp.sum(p1[:] * t[:, :_H], axis=1, keepdims=True)
               + jnp.sum(p2[:] * t[:, _H:], axis=1, keepdims=True) + bias)
    logit_n = (jnp.sum(n1[:] * t[:, :_H], axis=1, keepdims=True)
               + jnp.sum(n2[:] * t[:, _H:], axis=1, keepdims=True) + bias)
    sig = jax.nn.sigmoid(jnp.concatenate([logit_p, logit_n], axis=1))
    out_ref[:] = jnp.transpose(sig, (1, 0))[None]


def kernel(anchor_feat, pos_feat, neg_feat, W, b):
    n = anchor_feat.shape[0]
    w0 = W[0]
    g = n // _BLOCK

    left = pl.BlockSpec((_BLOCK, _H), lambda i: (i, 0))
    right = pl.BlockSpec((_BLOCK, _H), lambda i: (i, 1))
    w_spec = pl.BlockSpec((_D, _D), lambda i: (0, 0))
    b_spec = pl.BlockSpec(memory_space=pltpu.SMEM)
    out_spec = pl.BlockSpec((1, 2, _BLOCK), lambda i: (i, 0, 0))

    scores = pl.pallas_call(
        _bilinear_kernel,
        grid=(g,),
        in_specs=[left, right, left, right, left, right, w_spec, b_spec],
        out_specs=out_spec,
        out_shape=jax.ShapeDtypeStruct((g, 2, _BLOCK), jnp.float32),
        compiler_params=pltpu.CompilerParams(
            dimension_semantics=("parallel",),
            vmem_limit_bytes=128 * 1024 * 1024,
        ),
    )(anchor_feat, anchor_feat, pos_feat, pos_feat, neg_feat, neg_feat, w0, b)

    return (scores[:, 0, :].reshape(-1), scores[:, 1, :].reshape(-1))


# speedup vs baseline: 1.0156x; 1.0156x over previous
import jax
import jax.numpy as jnp
from jax.experimental import pallas as pl
from jax.experimental.pallas import tpu as pltpu

_D = 512
_BLOCK = 2000


def _bilinear_kernel(a_ref, p_ref, n_ref, w_ref, b_ref, out_ref):
    t = jax.lax.dot_general(a_ref[:], w_ref[:],
                            (((1,), (1,)), ((), ())),
                            preferred_element_type=jnp.float32)
    bias = b_ref[0]
    logit_p = jnp.sum(p_ref[:] * t, axis=1, keepdims=True)
    logit_n = jnp.sum(n_ref[:] * t, axis=1, keepdims=True)
    logits = jnp.transpose(jnp.concatenate([logit_p, logit_n], axis=1), (1, 0))
    out_ref[:] = jax.nn.sigmoid(logits + bias)[None]


def kernel(anchor_feat, pos_feat, neg_feat, W, b):
    n = anchor_feat.shape[0]
    w0 = W[0]
    g = n // _BLOCK

    grid = (g,)
    feat_spec = pl.BlockSpec((_BLOCK, _D), lambda i: (i, 0))
    w_spec = pl.BlockSpec((_D, _D), lambda i: (0, 0))
    b_spec = pl.BlockSpec(memory_space=pltpu.SMEM)
    out_spec = pl.BlockSpec((1, 2, _BLOCK), lambda i: (i, 0, 0))

    scores = pl.pallas_call(
        _bilinear_kernel,
        grid=grid,
        in_specs=[feat_spec, feat_spec, feat_spec, w_spec, b_spec],
        out_specs=out_spec,
        out_shape=jax.ShapeDtypeStruct((g, 2, _BLOCK), jnp.float32),
        compiler_params=pltpu.CompilerParams(
            dimension_semantics=("parallel",),
            vmem_limit_bytes=128 * 1024 * 1024,
        ),
    )(anchor_feat, pos_feat, neg_feat, w0, b)

    return (scores[:, 0, :].reshape(-1), scores[:, 1, :].reshape(-1))
